# baseline (speedup 1.0000x reference)
import functools

import jax
import jax.numpy as jnp
from jax import lax
from jax.experimental import pallas as pl
from jax.experimental.pallas import tpu as pltpu
from jax.experimental.pallas import tpu_sc as plsc

B, S, D, M, L = 2, 2048, 1024, 256, 16
SIM_K, CONT_K = 8, 4
MEM_LEN = (SIM_K + CONT_K) * L
N_ROWS = B * MEM_LEN
C_BLK = 64
N_CTX = MEM_LEN + S
S_BLK = 128
NEG = -3.0e38


def _stage1_body(q_ref, wsim_ref, bsim_ref, mkeys_ref, ts_ref,
                 idx_ref, acc_ref):
    i = pl.program_id(0)

    @pl.when(i == 0)
    def _():
        acc_ref[...] = jnp.zeros_like(acc_ref)

    acc_ref[...] += jnp.sum(q_ref[...], axis=1)

    @pl.when(i == pl.num_programs(0) - 1)
    def _():
        q_mean = acc_ref[...] * (1.0 / S)
        q_proj = lax.dot_general(
            q_mean, wsim_ref[...], (((1,), (1,)), ((), ())),
            preferred_element_type=jnp.float32) + bsim_ref[...]
        scores = lax.dot_general(
            q_proj, mkeys_ref[...], (((1,), (1,)), ((), ())),
            preferred_element_type=jnp.float32)

        iota = lax.broadcasted_iota(jnp.int32, (B, M), 1)
        s = scores
        sim_am = []
        for _ in range(SIM_K):
            m = jnp.max(s, axis=1, keepdims=True)
            am = jnp.min(jnp.where(s >= m, iota, M), axis=1, keepdims=True)
            sim_am.append(am)
            s = jnp.where(iota == am, NEG, s)

        t = ts_ref[...]
        tio = lax.broadcasted_iota(jnp.int32, (1, M), 1)
        cont_am = []
        for _ in range(CONT_K):
            m = jnp.max(t, axis=1, keepdims=True)
            am = jnp.min(jnp.where(t >= m, tio, M), axis=1, keepdims=True)
            cont_am.append(am)
            t = jnp.where(tio == am, NEG, t)

        p = lax.broadcasted_iota(jnp.int32, (1, N_ROWS), 1)
        grp = lax.shift_right_logical(p, 4)
        rows = jnp.bitwise_and(p, 15)
        for k in range(SIM_K):
            a0 = jnp.broadcast_to(sim_am[k][0:1, 0:1], (1, N_ROWS))
            a1 = jnp.broadcast_to(sim_am[k][1:2, 0:1], (1, N_ROWS))
            rows = rows + jnp.where(grp == k, a0 * L, 0)
            rows = rows + jnp.where(grp == SIM_K + CONT_K + k, a1 * L, 0)
        for k in range(CONT_K):
            c = jnp.broadcast_to(cont_am[k][0:1, 0:1], (1, N_ROWS))
            rows = rows + jnp.where(grp == SIM_K + k, c * L, 0)
            rows = rows + jnp.where(grp == 2 * SIM_K + CONT_K + k, c * L, 0)
        idx_ref[...] = rows


def _stage1(query, w_sim, b_sim, m_keys, ts):
    return pl.pallas_call(
        _stage1_body,
        grid=(S // S_BLK,),
        in_specs=[
            pl.BlockSpec((B, S_BLK, D), lambda i: (0, i, 0)),
            pl.BlockSpec((D, D), lambda i: (0, 0)),
            pl.BlockSpec((1, D), lambda i: (0, 0)),
            pl.BlockSpec((M, D), lambda i: (0, 0)),
            pl.BlockSpec((1, M), lambda i: (0, 0)),
        ],
        out_specs=pl.BlockSpec((1, N_ROWS), lambda i: (0, 0)),
        out_shape=jax.ShapeDtypeStruct((1, N_ROWS), jnp.int32),
        scratch_shapes=[pltpu.VMEM((B, D), jnp.float32)],
    )(query, w_sim, b_sim, m_keys, ts)


K_CHUNK = S // 16


@functools.cache
def _make_sc_gather_copy():
    mesh = plsc.VectorSubcoreMesh(core_axis_name="c", subcore_axis_name="s")

    @functools.partial(
        pl.kernel,
        mesh=mesh,
        out_type=[
            jax.ShapeDtypeStruct((N_ROWS, D), jnp.float32),
            jax.ShapeDtypeStruct((B, N_CTX, D), jnp.float32),
        ],
        scratch_types=[
            pltpu.VMEM((16,), jnp.int32),
            pltpu.VMEM((16, D), jnp.float32),
            pltpu.SemaphoreType.DMA,
            pltpu.SemaphoreType.DMA,
        ],
    )
    def _sc_gather_copy(table_hbm, idx_hbm, key_hbm, rows_hbm, ctx_hbm,
                        idx_v, rows_v, gsem, ksem):
        wid = lax.axis_index("s") * 2 + lax.axis_index("c")
        b = wid // 16
        r0 = (wid % 16) * K_CHUNK
        kcp = pltpu.make_async_copy(
            key_hbm.at[b, pl.ds(r0, K_CHUNK)],
            ctx_hbm.at[b, pl.ds(MEM_LEN + r0, K_CHUNK)],
            ksem)
        kcp.start()

        @pl.when(wid < N_ROWS // 16)
        def _():
            base = wid * 16
            pltpu.sync_copy(idx_hbm.at[pl.ds(base, 16)], idx_v)
            pltpu.async_copy(table_hbm.at[idx_v], rows_v, gsem).wait()
            pltpu.sync_copy(rows_v, rows_hbm.at[pl.ds(base, 16)])

        kcp.wait()

    return _sc_gather_copy


def _mlp_body(ctx_ref, x_ref, w1_ref, b1_ref, w2_ref, b2_ref, o_ref):
    del ctx_ref
    h = lax.dot_general(
        x_ref[...], w1_ref[...], (((1,), (1,)), ((), ())),
        preferred_element_type=jnp.float32) + b1_ref[...]
    h = jnp.maximum(h, 0.0)
    y = lax.dot_general(
        h, w2_ref[...], (((1,), (1,)), ((), ())),
        preferred_element_type=jnp.float32) + b2_ref[...]
    o_ref[...] = y.reshape(B, MEM_LEN, D)


def _mlp_into_ctx(ctx0, x, w1, b1, w2, b2):
    return pl.pallas_call(
        _mlp_body,
        grid=(1,),
        in_specs=[
            pl.BlockSpec((B, MEM_LEN, D), lambda i: (0, 0, 0)),
            pl.BlockSpec((N_ROWS, D), lambda i: (0, 0)),
            pl.BlockSpec((D, D), lambda i: (0, 0)),
            pl.BlockSpec((1, D), lambda i: (0, 0)),
            pl.BlockSpec((D, D), lambda i: (0, 0)),
            pl.BlockSpec((1, D), lambda i: (0, 0)),
        ],
        out_specs=pl.BlockSpec((B, MEM_LEN, D), lambda i: (0, 0, 0)),
        out_shape=jax.ShapeDtypeStruct((B, N_CTX, D), jnp.float32),
        input_output_aliases={0: 0},
    )(ctx0, x, w1, b1, w2, b2)


def kernel(query, key, value, attention_mask, W_brain, W_store1, b_store1,
           W_store2, b_store2, W_sim, b_sim, memory_keys, memory_values,
           memory_timestamps):
    idx = _stage1(query, W_sim, b_sim.reshape(1, D), memory_keys,
                  memory_timestamps.reshape(1, M))
    table = memory_values.reshape(M * L, D)
    rows, ctx0 = _make_sc_gather_copy()(table, idx.reshape(N_ROWS), key)
    context = _mlp_into_ctx(ctx0, rows, W_store1, b_store1.reshape(1, D),
                            W_store2, b_store2.reshape(1, D))
    ext_mask = jnp.concatenate(
        [jnp.ones((B, MEM_LEN), attention_mask.dtype), attention_mask], axis=1)
    return (query, context, context, ext_mask)


# speedup vs baseline: 7.8767x; 7.8767x over previous
import functools

import jax
import jax.numpy as jnp
from jax import lax
from jax.experimental import pallas as pl
from jax.experimental.pallas import tpu as pltpu
from jax.experimental.pallas import tpu_sc as plsc

B, S, D, M, L = 2, 2048, 1024, 256, 16
SIM_K, CONT_K = 8, 4
MEM_LEN = (SIM_K + CONT_K) * L
N_ROWS = B * MEM_LEN
C_BLK = 64
N_CTX = MEM_LEN + S
S_BLK = 128
NEG = -3.0e38


def _stage1_body(q_ref, wsim_ref, bsim_ref, mkeys_ref, ts_ref,
                 idx_ref, acc_ref):
    i = pl.program_id(0)

    @pl.when(i == 0)
    def _():
        acc_ref[...] = jnp.zeros_like(acc_ref)

    acc_ref[...] += jnp.sum(q_ref[...], axis=1)

    @pl.when(i == pl.num_programs(0) - 1)
    def _():
        q_mean = acc_ref[...] * (1.0 / S)
        q_proj = lax.dot_general(
            q_mean, wsim_ref[...], (((1,), (1,)), ((), ())),
            preferred_element_type=jnp.float32) + bsim_ref[...]
        scores = lax.dot_general(
            q_proj, mkeys_ref[...], (((1,), (1,)), ((), ())),
            preferred_element_type=jnp.float32)

        iota = lax.broadcasted_iota(jnp.int32, (B, M), 1)
        s = scores
        sim_am = []
        for _ in range(SIM_K):
            m = jnp.max(s, axis=1, keepdims=True)
            am = jnp.min(jnp.where(s >= m, iota, M), axis=1, keepdims=True)
            sim_am.append(am)
            s = jnp.where(iota == am, NEG, s)

        t = ts_ref[...]
        tio = lax.broadcasted_iota(jnp.int32, (1, M), 1)
        cont_am = []
        for _ in range(CONT_K):
            m = jnp.max(t, axis=1, keepdims=True)
            am = jnp.min(jnp.where(t >= m, tio, M), axis=1, keepdims=True)
            cont_am.append(am)
            t = jnp.where(tio == am, NEG, t)

        p = lax.broadcasted_iota(jnp.int32, (1, N_ROWS), 1)
        grp = lax.shift_right_logical(p, 4)
        rows = jnp.bitwise_and(p, 15)
        for k in range(SIM_K):
            a0 = jnp.broadcast_to(sim_am[k][0:1, 0:1], (1, N_ROWS))
            a1 = jnp.broadcast_to(sim_am[k][1:2, 0:1], (1, N_ROWS))
            rows = rows + jnp.where(grp == k, a0 * L, 0)
            rows = rows + jnp.where(grp == SIM_K + CONT_K + k, a1 * L, 0)
        for k in range(CONT_K):
            c = jnp.broadcast_to(cont_am[k][0:1, 0:1], (1, N_ROWS))
            rows = rows + jnp.where(grp == SIM_K + k, c * L, 0)
            rows = rows + jnp.where(grp == 2 * SIM_K + CONT_K + k, c * L, 0)
        idx_ref[...] = rows


def _stage1(query, w_sim, b_sim, m_keys, ts):
    return pl.pallas_call(
        _stage1_body,
        grid=(S // S_BLK,),
        in_specs=[
            pl.BlockSpec((B, S_BLK, D), lambda i: (0, i, 0)),
            pl.BlockSpec((D, D), lambda i: (0, 0)),
            pl.BlockSpec((1, D), lambda i: (0, 0)),
            pl.BlockSpec((M, D), lambda i: (0, 0)),
            pl.BlockSpec((1, M), lambda i: (0, 0)),
        ],
        out_specs=pl.BlockSpec((1, N_ROWS), lambda i: (0, 0)),
        out_shape=jax.ShapeDtypeStruct((1, N_ROWS), jnp.int32),
        scratch_shapes=[pltpu.VMEM((B, D), jnp.float32)],
    )(query, w_sim, b_sim, m_keys, ts)


@functools.cache
def _make_sc_gather():
    mesh = plsc.VectorSubcoreMesh(core_axis_name="c", subcore_axis_name="s")

    @functools.partial(
        pl.kernel,
        mesh=mesh,
        out_type=jax.ShapeDtypeStruct((N_ROWS, D), jnp.float32),
        scratch_types=[
            pltpu.VMEM((16,), jnp.int32),
            pltpu.VMEM((16, D), jnp.float32),
            pltpu.SemaphoreType.DMA,
        ],
    )
    def _sc_gather(table_hbm, idx_hbm, out_hbm, idx_v, rows_v, sem):
        wid = lax.axis_index("s") * 2 + lax.axis_index("c")

        @pl.when(wid < N_ROWS // 16)
        def _():
            base = wid * 16
            pltpu.sync_copy(idx_hbm.at[pl.ds(base, 16)], idx_v)
            pltpu.async_copy(table_hbm.at[idx_v], rows_v, sem).wait()
            pltpu.sync_copy(rows_v, out_hbm.at[pl.ds(base, 16)])

    return _sc_gather


def _mlp_body(x_ref, w1_ref, b1_ref, w2_ref, b2_ref, o_ref):
    h = lax.dot_general(
        x_ref[...], w1_ref[...], (((1,), (1,)), ((), ())),
        preferred_element_type=jnp.float32) + b1_ref[...]
    h = jnp.maximum(h, 0.0)
    o_ref[...] = lax.dot_general(
        h, w2_ref[...], (((1,), (1,)), ((), ())),
        preferred_element_type=jnp.float32) + b2_ref[...]


def _mlp(x, w1, b1, w2, b2):
    return pl.pallas_call(
        _mlp_body,
        out_shape=jax.ShapeDtypeStruct((x.shape[0], D), jnp.float32),
    )(x, w1, b1, w2, b2)


def kernel(query, key, value, attention_mask, W_brain, W_store1, b_store1,
           W_store2, b_store2, W_sim, b_sim, memory_keys, memory_values,
           memory_timestamps):
    idx = _stage1(query, W_sim, b_sim.reshape(1, D), memory_keys,
                  memory_timestamps.reshape(1, M))
    table = memory_values.reshape(M * L, D)
    rows = _make_sc_gather()(table, idx.reshape(N_ROWS))
    y = _mlp(rows, W_store1, b_store1.reshape(1, D),
             W_store2, b_store2.reshape(1, D))
    context = jnp.concatenate([y.reshape(B, MEM_LEN, D), key], axis=1)
    ext_mask = jnp.concatenate(
        [jnp.ones((B, MEM_LEN), attention_mask.dtype), attention_mask], axis=1)
    return (query, context, context, ext_mask)


# speedup vs baseline: 8.3270x; 1.0572x over previous
import functools

import jax
import jax.numpy as jnp
from jax import lax
from jax.experimental import pallas as pl
from jax.experimental.pallas import tpu as pltpu
from jax.experimental.pallas import tpu_sc as plsc

B, S, D, M, L = 2, 2048, 1024, 256, 16
SIM_K, CONT_K = 8, 4
MEM_LEN = (SIM_K + CONT_K) * L
N_ROWS = B * MEM_LEN
C_BLK = 64
N_CTX = MEM_LEN + S
S_BLK = 256
NEG = -3.0e38


def _stage1_body(q_ref, wsim_ref, bsim_ref, mkeys_ref, ts_ref,
                 idx_ref, acc_ref):
    i = pl.program_id(0)

    @pl.when(i == 0)
    def _():
        acc_ref[...] = jnp.zeros_like(acc_ref)

    acc_ref[...] += jnp.sum(q_ref[...], axis=1)

    @pl.when(i == pl.num_programs(0) - 1)
    def _():
        q_mean = acc_ref[...] * (1.0 / S)
        q_proj = lax.dot_general(
            q_mean, wsim_ref[...], (((1,), (1,)), ((), ())),
            preferred_element_type=jnp.float32) + bsim_ref[...]
        scores = lax.dot_general(
            q_proj, mkeys_ref[...], (((1,), (1,)), ((), ())),
            preferred_element_type=jnp.float32)

        iota = lax.broadcasted_iota(jnp.int32, (B, M), 1)
        s = scores
        sim_am = []
        for _ in range(SIM_K):
            m = jnp.max(s, axis=1, keepdims=True)
            am = jnp.min(jnp.where(s >= m, iota, M), axis=1, keepdims=True)
            sim_am.append(am)
            s = jnp.where(iota == am, NEG, s)

        t = ts_ref[...]
        tio = lax.broadcasted_iota(jnp.int32, (1, M), 1)
        cont_am = []
        for _ in range(CONT_K):
            m = jnp.max(t, axis=1, keepdims=True)
            am = jnp.min(jnp.where(t >= m, tio, M), axis=1, keepdims=True)
            cont_am.append(am)
            t = jnp.where(tio == am, NEG, t)

        p = lax.broadcasted_iota(jnp.int32, (1, N_ROWS), 1)
        grp = lax.shift_right_logical(p, 4)
        rows = jnp.bitwise_and(p, 15)
        for k in range(SIM_K):
            a0 = jnp.broadcast_to(sim_am[k][0:1, 0:1], (1, N_ROWS))
            a1 = jnp.broadcast_to(sim_am[k][1:2, 0:1], (1, N_ROWS))
            rows = rows + jnp.where(grp == k, a0 * L, 0)
            rows = rows + jnp.where(grp == SIM_K + CONT_K + k, a1 * L, 0)
        for k in range(CONT_K):
            c = jnp.broadcast_to(cont_am[k][0:1, 0:1], (1, N_ROWS))
            rows = rows + jnp.where(grp == SIM_K + k, c * L, 0)
            rows = rows + jnp.where(grp == 2 * SIM_K + CONT_K + k, c * L, 0)
        idx_ref[...] = rows


def _stage1(query, w_sim, b_sim, m_keys, ts):
    return pl.pallas_call(
        _stage1_body,
        grid=(S // S_BLK,),
        in_specs=[
            pl.BlockSpec((B, S_BLK, D), lambda i: (0, i, 0)),
            pl.BlockSpec((D, D), lambda i: (0, 0)),
            pl.BlockSpec((1, D), lambda i: (0, 0)),
            pl.BlockSpec((M, D), lambda i: (0, 0)),
            pl.BlockSpec((1, M), lambda i: (0, 0)),
        ],
        out_specs=pl.BlockSpec((1, N_ROWS), lambda i: (0, 0)),
        out_shape=jax.ShapeDtypeStruct((1, N_ROWS), jnp.int32),
        scratch_shapes=[pltpu.VMEM((B, D), jnp.float32)],
    )(query, w_sim, b_sim, m_keys, ts)


@functools.cache
def _make_sc_gather():
    mesh = plsc.VectorSubcoreMesh(core_axis_name="c", subcore_axis_name="s")

    @functools.partial(
        pl.kernel,
        mesh=mesh,
        out_type=jax.ShapeDtypeStruct((N_ROWS, D), jnp.float32),
        scratch_types=[
            pltpu.VMEM((16,), jnp.int32),
            pltpu.VMEM((16, D), jnp.float32),
            pltpu.SemaphoreType.DMA,
        ],
    )
    def _sc_gather(table_hbm, idx_hbm, out_hbm, idx_v, rows_v, sem):
        wid = lax.axis_index("s") * 2 + lax.axis_index("c")

        @pl.when(wid < N_ROWS // 16)
        def _():
            base = wid * 16
            pltpu.sync_copy(idx_hbm.at[pl.ds(base, 16)], idx_v)
            pltpu.async_copy(table_hbm.at[idx_v], rows_v, sem).wait()
            pltpu.sync_copy(rows_v, out_hbm.at[pl.ds(base, 16)])

    return _sc_gather


def _mlp_body(x_ref, w1_ref, b1_ref, w2_ref, b2_ref, o_ref):
    h = lax.dot_general(
        x_ref[...], w1_ref[...], (((1,), (1,)), ((), ())),
        preferred_element_type=jnp.float32) + b1_ref[...]
    h = jnp.maximum(h, 0.0)
    o_ref[...] = lax.dot_general(
        h, w2_ref[...], (((1,), (1,)), ((), ())),
        preferred_element_type=jnp.float32) + b2_ref[...]


def _mlp(x, w1, b1, w2, b2):
    return pl.pallas_call(
        _mlp_body,
        out_shape=jax.ShapeDtypeStruct((x.shape[0], D), jnp.float32),
    )(x, w1, b1, w2, b2)


def kernel(query, key, value, attention_mask, W_brain, W_store1, b_store1,
           W_store2, b_store2, W_sim, b_sim, memory_keys, memory_values,
           memory_timestamps):
    idx = _stage1(query, W_sim, b_sim.reshape(1, D), memory_keys,
                  memory_timestamps.reshape(1, M))
    table = memory_values.reshape(M * L, D)
    rows = _make_sc_gather()(table, idx.reshape(N_ROWS))
    y = _mlp(rows, W_store1, b_store1.reshape(1, D),
             W_store2, b_store2.reshape(1, D))
    context = jnp.concatenate([y.reshape(B, MEM_LEN, D), key], axis=1)
    ext_mask = jnp.concatenate(
        [jnp.ones((B, MEM_LEN), attention_mask.dtype), attention_mask], axis=1)
    return (query, context, context, ext_mask)


# speedup vs baseline: 8.4951x; 1.0202x over previous
import functools

import jax
import jax.numpy as jnp
from jax import lax
from jax.experimental import pallas as pl
from jax.experimental.pallas import tpu as pltpu
from jax.experimental.pallas import tpu_sc as plsc

B, S, D, M, L = 2, 2048, 1024, 256, 16
SIM_K, CONT_K = 8, 4
MEM_LEN = (SIM_K + CONT_K) * L
N_ROWS = B * MEM_LEN
C_BLK = 64
N_CTX = MEM_LEN + S
S_BLK = 512
NEG = -3.0e38


def _stage1_body(q_ref, wsim_ref, bsim_ref, mkeys_ref, ts_ref,
                 idx_ref, acc_ref):
    i = pl.program_id(0)

    @pl.when(i == 0)
    def _():
        acc_ref[...] = jnp.zeros_like(acc_ref)

    acc_ref[...] += jnp.sum(q_ref[...], axis=1)

    @pl.when(i == pl.num_programs(0) - 1)
    def _():
        q_mean = acc_ref[...] * (1.0 / S)
        q_proj = lax.dot_general(
            q_mean, wsim_ref[...], (((1,), (1,)), ((), ())),
            preferred_element_type=jnp.float32) + bsim_ref[...]
        scores = lax.dot_general(
            q_proj, mkeys_ref[...], (((1,), (1,)), ((), ())),
            preferred_element_type=jnp.float32)

        iota = lax.broadcasted_iota(jnp.int32, (B, M), 1)
        s = scores
        sim_am = []
        for _ in range(SIM_K):
            m = jnp.max(s, axis=1, keepdims=True)
            am = jnp.min(jnp.where(s >= m, iota, M), axis=1, keepdims=True)
            sim_am.append(am)
            s = jnp.where(iota == am, NEG, s)

        t = ts_ref[...]
        tio = lax.broadcasted_iota(jnp.int32, (1, M), 1)
        cont_am = []
        for _ in range(CONT_K):
            m = jnp.max(t, axis=1, keepdims=True)
            am = jnp.min(jnp.where(t >= m, tio, M), axis=1, keepdims=True)
            cont_am.append(am)
            t = jnp.where(tio == am, NEG, t)

        p = lax.broadcasted_iota(jnp.int32, (1, N_ROWS), 1)
        grp = lax.shift_right_logical(p, 4)
        rows = jnp.bitwise_and(p, 15)
        for k in range(SIM_K):
            a0 = jnp.broadcast_to(sim_am[k][0:1, 0:1], (1, N_ROWS))
            a1 = jnp.broadcast_to(sim_am[k][1:2, 0:1], (1, N_ROWS))
            rows = rows + jnp.where(grp == k, a0 * L, 0)
            rows = rows + jnp.where(grp == SIM_K + CONT_K + k, a1 * L, 0)
        for k in range(CONT_K):
            c = jnp.broadcast_to(cont_am[k][0:1, 0:1], (1, N_ROWS))
            rows = rows + jnp.where(grp == SIM_K + k, c * L, 0)
            rows = rows + jnp.where(grp == 2 * SIM_K + CONT_K + k, c * L, 0)
        idx_ref[...] = rows


def _stage1(query, w_sim, b_sim, m_keys, ts):
    return pl.pallas_call(
        _stage1_body,
        grid=(S // S_BLK,),
        in_specs=[
            pl.BlockSpec((B, S_BLK, D), lambda i: (0, i, 0)),
            pl.BlockSpec((D, D), lambda i: (0, 0)),
            pl.BlockSpec((1, D), lambda i: (0, 0)),
            pl.BlockSpec((M, D), lambda i: (0, 0)),
            pl.BlockSpec((1, M), lambda i: (0, 0)),
        ],
        out_specs=pl.BlockSpec((1, N_ROWS), lambda i: (0, 0)),
        out_shape=jax.ShapeDtypeStruct((1, N_ROWS), jnp.int32),
        scratch_shapes=[pltpu.VMEM((B, D), jnp.float32)],
    )(query, w_sim, b_sim, m_keys, ts)


@functools.cache
def _make_sc_gather():
    mesh = plsc.VectorSubcoreMesh(core_axis_name="c", subcore_axis_name="s")

    @functools.partial(
        pl.kernel,
        mesh=mesh,
        out_type=jax.ShapeDtypeStruct((N_ROWS, D), jnp.float32),
        scratch_types=[
            pltpu.VMEM((16,), jnp.int32),
            pltpu.VMEM((16, D), jnp.float32),
            pltpu.SemaphoreType.DMA,
        ],
    )
    def _sc_gather(table_hbm, idx_hbm, out_hbm, idx_v, rows_v, sem):
        wid = lax.axis_index("s") * 2 + lax.axis_index("c")

        @pl.when(wid < N_ROWS // 16)
        def _():
            base = wid * 16
            pltpu.sync_copy(idx_hbm.at[pl.ds(base, 16)], idx_v)
            pltpu.async_copy(table_hbm.at[idx_v], rows_v, sem).wait()
            pltpu.sync_copy(rows_v, out_hbm.at[pl.ds(base, 16)])

    return _sc_gather


def _mlp_body(x_ref, w1_ref, b1_ref, w2_ref, b2_ref, o_ref):
    h = lax.dot_general(
        x_ref[...], w1_ref[...], (((1,), (1,)), ((), ())),
        preferred_element_type=jnp.float32) + b1_ref[...]
    h = jnp.maximum(h, 0.0)
    o_ref[...] = lax.dot_general(
        h, w2_ref[...], (((1,), (1,)), ((), ())),
        preferred_element_type=jnp.float32) + b2_ref[...]


def _mlp(x, w1, b1, w2, b2):
    return pl.pallas_call(
        _mlp_body,
        out_shape=jax.ShapeDtypeStruct((x.shape[0], D), jnp.float32),
    )(x, w1, b1, w2, b2)


def kernel(query, key, value, attention_mask, W_brain, W_store1, b_store1,
           W_store2, b_store2, W_sim, b_sim, memory_keys, memory_values,
           memory_timestamps):
    idx = _stage1(query, W_sim, b_sim.reshape(1, D), memory_keys,
                  memory_timestamps.reshape(1, M))
    table = memory_values.reshape(M * L, D)
    rows = _make_sc_gather()(table, idx.reshape(N_ROWS))
    y = _mlp(rows, W_store1, b_store1.reshape(1, D),
             W_store2, b_store2.reshape(1, D))
    context = jnp.concatenate([y.reshape(B, MEM_LEN, D), key], axis=1)
    ext_mask = jnp.concatenate(
        [jnp.ones((B, MEM_LEN), attention_mask.dtype), attention_mask], axis=1)
    return (query, context, context, ext_mask)


# speedup vs baseline: 8.5097x; 1.0017x over previous
import functools

import jax
import jax.numpy as jnp
from jax import lax
from jax.experimental import pallas as pl
from jax.experimental.pallas import tpu as pltpu
from jax.experimental.pallas import tpu_sc as plsc

B, S, D, M, L = 2, 2048, 1024, 256, 16
SIM_K, CONT_K = 8, 4
MEM_LEN = (SIM_K + CONT_K) * L
N_ROWS = B * MEM_LEN
C_BLK = 64
N_CTX = MEM_LEN + S
S_BLK = 1024
NEG = -3.0e38


def _stage1_body(q_ref, wsim_ref, bsim_ref, mkeys_ref, ts_ref,
                 idx_ref, acc_ref):
    i = pl.program_id(0)

    @pl.when(i == 0)
    def _():
        acc_ref[...] = jnp.zeros_like(acc_ref)

    acc_ref[...] += jnp.sum(q_ref[...], axis=1)

    @pl.when(i == pl.num_programs(0) - 1)
    def _():
        q_mean = acc_ref[...] * (1.0 / S)
        q_proj = lax.dot_general(
            q_mean, wsim_ref[...], (((1,), (1,)), ((), ())),
            preferred_element_type=jnp.float32) + bsim_ref[...]
        scores = lax.dot_general(
            q_proj, mkeys_ref[...], (((1,), (1,)), ((), ())),
            preferred_element_type=jnp.float32)

        iota = lax.broadcasted_iota(jnp.int32, (B, M), 1)
        s = scores
        sim_am = []
        for _ in range(SIM_K):
            m = jnp.max(s, axis=1, keepdims=True)
            am = jnp.min(jnp.where(s >= m, iota, M), axis=1, keepdims=True)
            sim_am.append(am)
            s = jnp.where(iota == am, NEG, s)

        t = ts_ref[...]
        tio = lax.broadcasted_iota(jnp.int32, (1, M), 1)
        cont_am = []
        for _ in range(CONT_K):
            m = jnp.max(t, axis=1, keepdims=True)
            am = jnp.min(jnp.where(t >= m, tio, M), axis=1, keepdims=True)
            cont_am.append(am)
            t = jnp.where(tio == am, NEG, t)

        p = lax.broadcasted_iota(jnp.int32, (1, N_ROWS), 1)
        grp = lax.shift_right_logical(p, 4)
        rows = jnp.bitwise_and(p, 15)
        for k in range(SIM_K):
            a0 = jnp.broadcast_to(sim_am[k][0:1, 0:1], (1, N_ROWS))
            a1 = jnp.broadcast_to(sim_am[k][1:2, 0:1], (1, N_ROWS))
            rows = rows + jnp.where(grp == k, a0 * L, 0)
            rows = rows + jnp.where(grp == SIM_K + CONT_K + k, a1 * L, 0)
        for k in range(CONT_K):
            c = jnp.broadcast_to(cont_am[k][0:1, 0:1], (1, N_ROWS))
            rows = rows + jnp.where(grp == SIM_K + k, c * L, 0)
            rows = rows + jnp.where(grp == 2 * SIM_K + CONT_K + k, c * L, 0)
        idx_ref[...] = rows


def _stage1(query, w_sim, b_sim, m_keys, ts):
    return pl.pallas_call(
        _stage1_body,
        grid=(S // S_BLK,),
        in_specs=[
            pl.BlockSpec((B, S_BLK, D), lambda i: (0, i, 0)),
            pl.BlockSpec((D, D), lambda i: (0, 0)),
            pl.BlockSpec((1, D), lambda i: (0, 0)),
            pl.BlockSpec((M, D), lambda i: (0, 0)),
            pl.BlockSpec((1, M), lambda i: (0, 0)),
        ],
        out_specs=pl.BlockSpec((1, N_ROWS), lambda i: (0, 0)),
        out_shape=jax.ShapeDtypeStruct((1, N_ROWS), jnp.int32),
        scratch_shapes=[pltpu.VMEM((B, D), jnp.float32)],
    )(query, w_sim, b_sim, m_keys, ts)


@functools.cache
def _make_sc_gather():
    mesh = plsc.VectorSubcoreMesh(core_axis_name="c", subcore_axis_name="s")

    @functools.partial(
        pl.kernel,
        mesh=mesh,
        out_type=jax.ShapeDtypeStruct((N_ROWS, D), jnp.float32),
        scratch_types=[
            pltpu.VMEM((16,), jnp.int32),
            pltpu.VMEM((16, D), jnp.float32),
            pltpu.SemaphoreType.DMA,
        ],
    )
    def _sc_gather(table_hbm, idx_hbm, out_hbm, idx_v, rows_v, sem):
        wid = lax.axis_index("s") * 2 + lax.axis_index("c")

        @pl.when(wid < N_ROWS // 16)
        def _():
            base = wid * 16
            pltpu.sync_copy(idx_hbm.at[pl.ds(base, 16)], idx_v)
            pltpu.async_copy(table_hbm.at[idx_v], rows_v, sem).wait()
            pltpu.sync_copy(rows_v, out_hbm.at[pl.ds(base, 16)])

    return _sc_gather


def _mlp_body(x_ref, w1_ref, b1_ref, w2_ref, b2_ref, o_ref):
    h = lax.dot_general(
        x_ref[...], w1_ref[...], (((1,), (1,)), ((), ())),
        preferred_element_type=jnp.float32) + b1_ref[...]
    h = jnp.maximum(h, 0.0)
    o_ref[...] = lax.dot_general(
        h, w2_ref[...], (((1,), (1,)), ((), ())),
        preferred_element_type=jnp.float32) + b2_ref[...]


def _mlp(x, w1, b1, w2, b2):
    return pl.pallas_call(
        _mlp_body,
        out_shape=jax.ShapeDtypeStruct((x.shape[0], D), jnp.float32),
    )(x, w1, b1, w2, b2)


def kernel(query, key, value, attention_mask, W_brain, W_store1, b_store1,
           W_store2, b_store2, W_sim, b_sim, memory_keys, memory_values,
           memory_timestamps):
    idx = _stage1(query, W_sim, b_sim.reshape(1, D), memory_keys,
                  memory_timestamps.reshape(1, M))
    table = memory_values.reshape(M * L, D)
    rows = _make_sc_gather()(table, idx.reshape(N_ROWS))
    y = _mlp(rows, W_store1, b_store1.reshape(1, D),
             W_store2, b_store2.reshape(1, D))
    context = jnp.concatenate([y.reshape(B, MEM_LEN, D), key], axis=1)
    ext_mask = jnp.concatenate(
        [jnp.ones((B, MEM_LEN), attention_mask.dtype), attention_mask], axis=1)
    return (query, context, context, ext_mask)
